# baseline (speedup 1.0000x reference)
import functools

import jax
import jax.numpy as jnp
from jax import lax
from jax.experimental import pallas as pl
from jax.experimental.pallas import tpu as pltpu
from jax.experimental.pallas import tpu_sc as plsc

_V = 1000000
_EMB = 64
_ROW = 128
_CLS = 128
_B = 4096
_T = 200

_NC, _NS = 2, 16
_NW = _NC * _NS
_BPW = _B // _NW
_C0 = 104
_C1 = _T - _C0

_mesh = plsc.VectorSubcoreMesh(core_axis_name="c", subcore_axis_name="s")


@functools.partial(
    pl.kernel,
    mesh=_mesh,
    compiler_params=pltpu.CompilerParams(use_tc_tiling_on_sc=False),
    out_type=jax.ShapeDtypeStruct((_B, _EMB), jnp.float32),
    scratch_types=[
        pltpu.VMEM((_BPW, _T), jnp.int32),
        pltpu.VMEM((_T, _ROW), jnp.float32),
        pltpu.VMEM((_BPW, _EMB), jnp.float32),
        pltpu.SemaphoreType.DMA,
        pltpu.SemaphoreType.DMA,
    ],
)
def _sc_sum_pool(x_hbm, table_hbm, out_hbm, idx_v, rows_v, pooled_v,
                 sem0, sem1):
    wid = lax.axis_index("s") * _NC + lax.axis_index("c")
    row0 = wid * _BPW
    pltpu.sync_copy(x_hbm.at[pl.ds(row0, _BPW)], idx_v)

    def body(i, carry):
        cp0 = pltpu.async_copy(
            table_hbm.at[idx_v.at[i, pl.ds(0, _C0)]], rows_v.at[pl.ds(0, _C0)],
            sem0)
        cp1 = pltpu.async_copy(
            table_hbm.at[idx_v.at[i, pl.ds(_C0, _C1)]],
            rows_v.at[pl.ds(_C0, _C1)], sem1)
        cp0.wait()
        cp1.wait()

        def rbody(t, accs):
            a0, a1, a2, a3 = accs
            return (a0 + rows_v[t, pl.ds(0, 16)],
                    a1 + rows_v[t, pl.ds(16, 16)],
                    a2 + rows_v[t, pl.ds(32, 16)],
                    a3 + rows_v[t, pl.ds(48, 16)])

        z = jnp.zeros((16,), jnp.float32)
        a0, a1, a2, a3 = lax.fori_loop(0, _T, rbody, (z, z, z, z))
        pooled_v[i, pl.ds(0, 16)] = a0
        pooled_v[i, pl.ds(16, 16)] = a1
        pooled_v[i, pl.ds(32, 16)] = a2
        pooled_v[i, pl.ds(48, 16)] = a3
        return carry

    lax.fori_loop(0, _BPW, body, 0)
    pltpu.sync_copy(pooled_v, out_hbm.at[pl.ds(row0, _BPW)])


def _mm_body(p_ref, w_ref, b_ref, o_ref):
    o_ref[...] = (jnp.dot(p_ref[...], w_ref[...],
                          preferred_element_type=jnp.float32) * (1.0 / _T)
                  + b_ref[...])


_TBLK = 2048


def _fmt_body(in_ref, o_ref):
    eye = jnp.eye(_EMB, dtype=jnp.float32)
    t = jax.lax.dot_general(in_ref[...], eye, (((0,), (0,)), ((), ())),
                            preferred_element_type=jnp.float32)
    o_ref[:, 0:_EMB] = t
    o_ref[:, _EMB:_ROW] = jnp.zeros((_TBLK, _ROW - _EMB), jnp.float32)


def _format_table(table):
    return pl.pallas_call(
        _fmt_body,
        grid=(pl.cdiv(_V, _TBLK),),
        in_specs=[pl.BlockSpec((_EMB, _TBLK), lambda i: (0, i))],
        out_specs=pl.BlockSpec((_TBLK, _ROW), lambda i: (i, 0)),
        out_shape=jax.ShapeDtypeStruct((_V, _ROW), jnp.float32),
    )(table.T)


def kernel(x, table, W, b):
    tblp = _format_table(table)
    pooled = _sc_sum_pool(x.astype(jnp.int32), tblp)
    logits = pl.pallas_call(
        _mm_body,
        out_shape=jax.ShapeDtypeStruct((_B, _CLS), jnp.float32),
    )(pooled, W, b.reshape(1, _CLS))
    return logits


# speedup vs baseline: 1.1834x; 1.1834x over previous
import functools

import jax
import jax.numpy as jnp
from jax import lax
from jax.experimental import pallas as pl
from jax.experimental.pallas import tpu as pltpu
from jax.experimental.pallas import tpu_sc as plsc

_V = 1000000
_EMB = 64
_ROW = 128
_CLS = 128
_B = 4096
_T = 200

_NC, _NS = 2, 16
_NW = _NC * _NS
_BPW = _B // _NW
_C0 = 104
_C1 = _T - _C0

_mesh = plsc.VectorSubcoreMesh(core_axis_name="c", subcore_axis_name="s")


@functools.partial(
    pl.kernel,
    mesh=_mesh,
    compiler_params=pltpu.CompilerParams(use_tc_tiling_on_sc=False),
    out_type=jax.ShapeDtypeStruct((_B, _EMB), jnp.float32),
    scratch_types=[
        pltpu.VMEM((_BPW, _T), jnp.int32),
        pltpu.VMEM((_T, _ROW), jnp.float32),
        pltpu.VMEM((_T, _ROW), jnp.float32),
        pltpu.VMEM((_BPW, _EMB), jnp.float32),
        pltpu.SemaphoreType.DMA,
        pltpu.SemaphoreType.DMA,
    ],
)
def _sc_sum_pool(x_hbm, table_hbm, out_hbm, idx_v, rows0_v, rows1_v,
                 pooled_v, sem0, sem1):
    wid = lax.axis_index("s") * _NC + lax.axis_index("c")
    row0 = wid * _BPW
    pltpu.sync_copy(x_hbm.at[pl.ds(row0, _BPW)], idx_v)

    def enq(i, rows_v, sem):
        pltpu.async_copy(
            table_hbm.at[idx_v.at[i, pl.ds(0, _C0)]], rows_v.at[pl.ds(0, _C0)],
            sem)
        pltpu.async_copy(
            table_hbm.at[idx_v.at[i, pl.ds(_C0, _C1)]],
            rows_v.at[pl.ds(_C0, _C1)], sem)

    def drain(i, rows_v, sem):
        pltpu.make_async_copy(
            table_hbm.at[idx_v.at[i, pl.ds(0, _C0)]], rows_v.at[pl.ds(0, _C0)],
            sem).wait()
        pltpu.make_async_copy(
            table_hbm.at[idx_v.at[i, pl.ds(_C0, _C1)]],
            rows_v.at[pl.ds(_C0, _C1)], sem).wait()

    def accum(i, rows_v):
        def rbody(u, accs):
            a0, a1, a2, a3 = accs
            t0 = pl.multiple_of(u * 8, 8)
            for r in range(8):
                t = t0 + r
                a0 = a0 + rows_v[t, pl.ds(0, 16)]
                a1 = a1 + rows_v[t, pl.ds(16, 16)]
                a2 = a2 + rows_v[t, pl.ds(32, 16)]
                a3 = a3 + rows_v[t, pl.ds(48, 16)]
            return (a0, a1, a2, a3)

        z = jnp.zeros((16,), jnp.float32)
        a0, a1, a2, a3 = lax.fori_loop(0, _T // 8, rbody, (z, z, z, z))
        pooled_v[i, pl.ds(0, 16)] = a0
        pooled_v[i, pl.ds(16, 16)] = a1
        pooled_v[i, pl.ds(32, 16)] = a2
        pooled_v[i, pl.ds(48, 16)] = a3

    enq(0, rows0_v, sem0)

    def body(g, carry):
        e0 = g * 2
        enq(e0 + 1, rows1_v, sem1)
        drain(e0, rows0_v, sem0)
        accum(e0, rows0_v)
        enq(jnp.minimum(e0 + 2, _BPW - 1), rows0_v, sem0)
        drain(e0 + 1, rows1_v, sem1)
        accum(e0 + 1, rows1_v)
        return carry

    lax.fori_loop(0, _BPW // 2, body, 0)
    drain(_BPW - 1, rows0_v, sem0)
    pltpu.sync_copy(pooled_v, out_hbm.at[pl.ds(row0, _BPW)])


def _mm_body(p_ref, w_ref, b_ref, o_ref):
    o_ref[...] = (jnp.dot(p_ref[...], w_ref[...],
                          preferred_element_type=jnp.float32) * (1.0 / _T)
                  + b_ref[...])


_TBLK = 2048


def _fmt_body(in_ref, o_ref):
    o_ref[:, 0:_EMB] = jnp.transpose(in_ref[...])
    o_ref[:, _EMB:_ROW] = jnp.zeros((_TBLK, _ROW - _EMB), jnp.float32)


def _format_table(table):
    return pl.pallas_call(
        _fmt_body,
        grid=(pl.cdiv(_V, _TBLK),),
        in_specs=[pl.BlockSpec((_EMB, _TBLK), lambda i: (0, i))],
        out_specs=pl.BlockSpec((_TBLK, _ROW), lambda i: (i, 0)),
        out_shape=jax.ShapeDtypeStruct((_V, _ROW), jnp.float32),
    )(table.T)


def kernel(x, table, W, b):
    tblp = _format_table(table)
    pooled = _sc_sum_pool(x.astype(jnp.int32), tblp)
    logits = pl.pallas_call(
        _mm_body,
        out_shape=jax.ShapeDtypeStruct((_B, _CLS), jnp.float32),
    )(pooled, W, b.reshape(1, _CLS))
    return logits


# speedup vs baseline: 1.6535x; 1.3973x over previous
import functools

import jax
import jax.numpy as jnp
from jax import lax
from jax.experimental import pallas as pl
from jax.experimental.pallas import tpu as pltpu
from jax.experimental.pallas import tpu_sc as plsc

_V = 1000000
_EMB = 64
_ROW = 128
_CLS = 128
_B = 4096
_T = 200

_NC, _NS = 2, 16
_NW = _NC * _NS
_BPW = _B // _NW
_C0 = 104
_C1 = _T - _C0

_mesh = plsc.VectorSubcoreMesh(core_axis_name="c", subcore_axis_name="s")


@functools.partial(
    pl.kernel,
    mesh=_mesh,
    compiler_params=pltpu.CompilerParams(use_tc_tiling_on_sc=False),
    out_type=jax.ShapeDtypeStruct((_B, _EMB), jnp.float32),
    scratch_types=[
        pltpu.VMEM((_BPW, _T), jnp.int32),
        pltpu.VMEM((_T, _ROW), jnp.float32),
        pltpu.VMEM((_T, _ROW), jnp.float32),
        pltpu.VMEM((_BPW, _EMB), jnp.float32),
        pltpu.SemaphoreType.DMA,
        pltpu.SemaphoreType.DMA,
    ],
)
def _sc_sum_pool(x_hbm, table_hbm, out_hbm, idx_v, rows0_v, rows1_v,
                 pooled_v, sem0, sem1):
    wid = lax.axis_index("s") * _NC + lax.axis_index("c")
    row0 = wid * _BPW
    pltpu.sync_copy(x_hbm.at[pl.ds(row0, _BPW)], idx_v)

    def enq(i, rows_v, sem):
        pltpu.async_copy(
            table_hbm.at[idx_v.at[i, pl.ds(0, _C0)]], rows_v.at[pl.ds(0, _C0)],
            sem)
        pltpu.async_copy(
            table_hbm.at[idx_v.at[i, pl.ds(_C0, _C1)]],
            rows_v.at[pl.ds(_C0, _C1)], sem)

    def drain(i, rows_v, sem):
        pltpu.make_async_copy(
            table_hbm.at[idx_v.at[i, pl.ds(0, _C0)]], rows_v.at[pl.ds(0, _C0)],
            sem).wait()
        pltpu.make_async_copy(
            table_hbm.at[idx_v.at[i, pl.ds(_C0, _C1)]],
            rows_v.at[pl.ds(_C0, _C1)], sem).wait()

    def accum(i, rows_v):
        def rbody(u, accs):
            a0, a1, a2, a3 = accs
            t0 = pl.multiple_of(u * 8, 8)
            for r in range(8):
                t = t0 + r
                a0 = a0 + rows_v[t, pl.ds(0, 16)]
                a1 = a1 + rows_v[t, pl.ds(16, 16)]
                a2 = a2 + rows_v[t, pl.ds(32, 16)]
                a3 = a3 + rows_v[t, pl.ds(48, 16)]
            return (a0, a1, a2, a3)

        z = jnp.zeros((16,), jnp.float32)
        a0, a1, a2, a3 = lax.fori_loop(0, _T // 8, rbody, (z, z, z, z))
        pooled_v[i, pl.ds(0, 16)] = a0
        pooled_v[i, pl.ds(16, 16)] = a1
        pooled_v[i, pl.ds(32, 16)] = a2
        pooled_v[i, pl.ds(48, 16)] = a3

    enq(0, rows0_v, sem0)

    def body(g, carry):
        e0 = g * 2
        enq(e0 + 1, rows1_v, sem1)
        drain(e0, rows0_v, sem0)
        accum(e0, rows0_v)
        enq(jnp.minimum(e0 + 2, _BPW - 1), rows0_v, sem0)
        drain(e0 + 1, rows1_v, sem1)
        accum(e0 + 1, rows1_v)
        return carry

    lax.fori_loop(0, _BPW // 2, body, 0)
    drain(_BPW - 1, rows0_v, sem0)
    pltpu.sync_copy(pooled_v, out_hbm.at[pl.ds(row0, _BPW)])


def _mm_body(p_ref, w_ref, b_ref, o_ref):
    o_ref[...] = (jnp.dot(p_ref[...], w_ref[...],
                          preferred_element_type=jnp.float32) * (1.0 / _T)
                  + b_ref[...])


_TBLK = 8192


def _fmt_body(in_ref, o_ref):
    o_ref[:, 0:_EMB] = jnp.transpose(in_ref[...])
    o_ref[:, _EMB:_ROW] = jnp.zeros((_TBLK, _ROW - _EMB), jnp.float32)


def _format_table(table):
    return pl.pallas_call(
        _fmt_body,
        grid=(pl.cdiv(_V, _TBLK),),
        in_specs=[pl.BlockSpec((_EMB, _TBLK), lambda i: (0, i))],
        out_specs=pl.BlockSpec((_TBLK, _ROW), lambda i: (i, 0)),
        out_shape=jax.ShapeDtypeStruct((_V, _ROW), jnp.float32),
    )(table.T)


def kernel(x, table, W, b):
    tblp = _format_table(table)
    pooled = _sc_sum_pool(x.astype(jnp.int32), tblp)
    logits = pl.pallas_call(
        _mm_body,
        out_shape=jax.ShapeDtypeStruct((_B, _CLS), jnp.float32),
    )(pooled, W, b.reshape(1, _CLS))
    return logits


# speedup vs baseline: 1.8188x; 1.1000x over previous
import functools

import jax
import jax.numpy as jnp
from jax import lax
from jax.experimental import pallas as pl
from jax.experimental.pallas import tpu as pltpu
from jax.experimental.pallas import tpu_sc as plsc

_V = 1000000
_EMB = 64
_ROW = 128
_CLS = 128
_B = 4096
_T = 200

_NC, _NS = 2, 16
_NW = _NC * _NS
_BPW = _B // _NW
_C0 = 104
_C1 = _T - _C0

_mesh = plsc.VectorSubcoreMesh(core_axis_name="c", subcore_axis_name="s")


@functools.partial(
    pl.kernel,
    mesh=_mesh,
    compiler_params=pltpu.CompilerParams(use_tc_tiling_on_sc=False),
    out_type=jax.ShapeDtypeStruct((_B, _EMB), jnp.float32),
    scratch_types=[
        pltpu.VMEM((_BPW, _T), jnp.int32),
        pltpu.VMEM((_T, _ROW), jnp.float32),
        pltpu.VMEM((_T, _ROW), jnp.float32),
        pltpu.VMEM((_T, _ROW), jnp.float32),
        pltpu.VMEM((_BPW, _EMB), jnp.float32),
        pltpu.SemaphoreType.DMA,
        pltpu.SemaphoreType.DMA,
        pltpu.SemaphoreType.DMA,
    ],
)
def _sc_sum_pool(x_hbm, table_hbm, out_hbm, idx_v, rows0_v, rows1_v,
                 rows2_v, pooled_v, sem0, sem1, sem2):
    wid = lax.axis_index("s") * _NC + lax.axis_index("c")
    row0 = wid * _BPW
    pltpu.sync_copy(x_hbm.at[pl.ds(row0, _BPW)], idx_v)

    def enq(i, rows_v, sem):
        pltpu.async_copy(
            table_hbm.at[idx_v.at[i, pl.ds(0, _C0)]], rows_v.at[pl.ds(0, _C0)],
            sem)
        pltpu.async_copy(
            table_hbm.at[idx_v.at[i, pl.ds(_C0, _C1)]],
            rows_v.at[pl.ds(_C0, _C1)], sem)

    def drain(i, rows_v, sem):
        pltpu.make_async_copy(
            table_hbm.at[idx_v.at[i, pl.ds(0, _C0)]], rows_v.at[pl.ds(0, _C0)],
            sem).wait()
        pltpu.make_async_copy(
            table_hbm.at[idx_v.at[i, pl.ds(_C0, _C1)]],
            rows_v.at[pl.ds(_C0, _C1)], sem).wait()

    def accum(i, rows_v):
        def rbody(u, accs):
            a0, a1, a2, a3 = accs
            t0 = pl.multiple_of(u * 8, 8)
            for r in range(8):
                t = t0 + r
                a0 = a0 + rows_v[t, pl.ds(0, 16)]
                a1 = a1 + rows_v[t, pl.ds(16, 16)]
                a2 = a2 + rows_v[t, pl.ds(32, 16)]
                a3 = a3 + rows_v[t, pl.ds(48, 16)]
            return (a0, a1, a2, a3)

        z = jnp.zeros((16,), jnp.float32)
        a0, a1, a2, a3 = lax.fori_loop(0, _T // 8, rbody, (z, z, z, z))
        pooled_v[i, pl.ds(0, 16)] = a0
        pooled_v[i, pl.ds(16, 16)] = a1
        pooled_v[i, pl.ds(32, 16)] = a2
        pooled_v[i, pl.ds(48, 16)] = a3

    slots = ((rows0_v, sem0), (rows1_v, sem1), (rows2_v, sem2))
    enq(0, rows0_v, sem0)
    enq(1, rows1_v, sem1)

    def body(g, carry):
        e0 = g * 3
        for k in range(3):
            rv, sm = slots[k]
            nrv, nsm = slots[(k + 2) % 3]
            enq(jnp.minimum(e0 + k + 2, _BPW - 1), nrv, nsm)
            drain(e0 + k, rv, sm)
            accum(e0 + k, rv)
        return carry

    lax.fori_loop(0, 42, body, 0)
    drain(126, rows0_v, sem0)
    accum(126, rows0_v)
    drain(127, rows1_v, sem1)
    accum(127, rows1_v)
    pltpu.sync_copy(pooled_v, out_hbm.at[pl.ds(row0, _BPW)])


def _mm_body(p_ref, w_ref, b_ref, o_ref):
    o_ref[...] = (jnp.dot(p_ref[...], w_ref[...],
                          preferred_element_type=jnp.float32) * (1.0 / _T)
                  + b_ref[...])


_TBLK = 16384


def _fmt_body(in_ref, o_ref):
    o_ref[:, 0:_EMB] = jnp.transpose(in_ref[...])
    o_ref[:, _EMB:_ROW] = jnp.zeros((_TBLK, _ROW - _EMB), jnp.float32)


def _format_table(table):
    return pl.pallas_call(
        _fmt_body,
        grid=(pl.cdiv(_V, _TBLK),),
        in_specs=[pl.BlockSpec((_EMB, _TBLK), lambda i: (0, i))],
        out_specs=pl.BlockSpec((_TBLK, _ROW), lambda i: (i, 0)),
        out_shape=jax.ShapeDtypeStruct((_V, _ROW), jnp.float32),
    )(table.T)


def kernel(x, table, W, b):
    tblp = _format_table(table)
    pooled = _sc_sum_pool(x.astype(jnp.int32), tblp)
    logits = pl.pallas_call(
        _mm_body,
        out_shape=jax.ShapeDtypeStruct((_B, _CLS), jnp.float32),
    )(pooled, W, b.reshape(1, _CLS))
    return logits


# speedup vs baseline: 1.8427x; 1.0131x over previous
import functools

import jax
import jax.numpy as jnp
from jax import lax
from jax.experimental import pallas as pl
from jax.experimental.pallas import tpu as pltpu
from jax.experimental.pallas import tpu_sc as plsc

_V = 1000000
_EMB = 64
_ROW = 128
_CLS = 128
_B = 4096
_T = 200

_NC, _NS = 2, 16
_NW = _NC * _NS
_BPW = _B // _NW
_C0 = 104
_C1 = _T - _C0

_mesh = plsc.VectorSubcoreMesh(core_axis_name="c", subcore_axis_name="s")


@functools.partial(
    pl.kernel,
    mesh=_mesh,
    compiler_params=pltpu.CompilerParams(use_tc_tiling_on_sc=False),
    out_type=jax.ShapeDtypeStruct((_B, _EMB), jnp.float32),
    scratch_types=[
        pltpu.VMEM((_BPW, _T), jnp.int32),
        pltpu.VMEM((_T, _ROW), jnp.float32),
        pltpu.VMEM((_T, _ROW), jnp.float32),
        pltpu.VMEM((_T, _ROW), jnp.float32),
        pltpu.VMEM((_BPW, _EMB), jnp.float32),
        pltpu.SemaphoreType.DMA,
        pltpu.SemaphoreType.DMA,
        pltpu.SemaphoreType.DMA,
    ],
)
def _sc_sum_pool(x_hbm, table_hbm, out_hbm, idx_v, rows0_v, rows1_v,
                 rows2_v, pooled_v, sem0, sem1, sem2):
    wid = lax.axis_index("s") * _NC + lax.axis_index("c")
    row0 = wid * _BPW
    pltpu.sync_copy(x_hbm.at[pl.ds(row0, _BPW)], idx_v)

    def enq(i, rows_v, sem):
        pltpu.async_copy(
            table_hbm.at[idx_v.at[i, pl.ds(0, _C0)]], rows_v.at[pl.ds(0, _C0)],
            sem)
        pltpu.async_copy(
            table_hbm.at[idx_v.at[i, pl.ds(_C0, _C1)]],
            rows_v.at[pl.ds(_C0, _C1)], sem)

    def drain(i, rows_v, sem):
        pltpu.make_async_copy(
            table_hbm.at[idx_v.at[i, pl.ds(0, _C0)]], rows_v.at[pl.ds(0, _C0)],
            sem).wait()
        pltpu.make_async_copy(
            table_hbm.at[idx_v.at[i, pl.ds(_C0, _C1)]],
            rows_v.at[pl.ds(_C0, _C1)], sem).wait()

    def accum(i, rows_v):
        def rbody(u, accs):
            a0, a1, a2, a3 = accs
            t0 = pl.multiple_of(u * 8, 8)
            for r in range(8):
                t = t0 + r
                a0 = a0 + rows_v[t, pl.ds(0, 16)]
                a1 = a1 + rows_v[t, pl.ds(16, 16)]
                a2 = a2 + rows_v[t, pl.ds(32, 16)]
                a3 = a3 + rows_v[t, pl.ds(48, 16)]
            return (a0, a1, a2, a3)

        z = jnp.zeros((16,), jnp.float32)
        a0, a1, a2, a3 = lax.fori_loop(0, _T // 8, rbody, (z, z, z, z))
        pooled_v[i, pl.ds(0, 16)] = a0
        pooled_v[i, pl.ds(16, 16)] = a1
        pooled_v[i, pl.ds(32, 16)] = a2
        pooled_v[i, pl.ds(48, 16)] = a3

    slots = ((rows0_v, sem0), (rows1_v, sem1), (rows2_v, sem2))
    enq(0, rows0_v, sem0)
    enq(1, rows1_v, sem1)

    def body(g, carry):
        e0 = g * 3
        for k in range(3):
            rv, sm = slots[k]
            nrv, nsm = slots[(k + 2) % 3]
            enq(jnp.minimum(e0 + k + 2, _BPW - 1), nrv, nsm)
            drain(e0 + k, rv, sm)
            accum(e0 + k, rv)
        return carry

    lax.fori_loop(0, 42, body, 0)
    drain(126, rows0_v, sem0)
    accum(126, rows0_v)
    drain(127, rows1_v, sem1)
    accum(127, rows1_v)
    pltpu.sync_copy(pooled_v, out_hbm.at[pl.ds(row0, _BPW)])


def _mm_body(p_ref, w_ref, b_ref, o_ref):
    o_ref[...] = (jnp.dot(p_ref[...], w_ref[...],
                          preferred_element_type=jnp.float32) * (1.0 / _T)
                  + b_ref[...])


_TBLK = 32768


def _fmt_body(in_ref, o_ref):
    o_ref[:, 0:_EMB] = jnp.transpose(in_ref[...])
    o_ref[:, _EMB:_ROW] = jnp.zeros((_TBLK, _ROW - _EMB), jnp.float32)


def _format_table(table):
    return pl.pallas_call(
        _fmt_body,
        grid=(pl.cdiv(_V, _TBLK),),
        in_specs=[pl.BlockSpec((_EMB, _TBLK), lambda i: (0, i))],
        out_specs=pl.BlockSpec((_TBLK, _ROW), lambda i: (i, 0)),
        out_shape=jax.ShapeDtypeStruct((_V, _ROW), jnp.float32),
    )(table.T)


def kernel(x, table, W, b):
    tblp = _format_table(table)
    pooled = _sc_sum_pool(x.astype(jnp.int32), tblp)
    logits = pl.pallas_call(
        _mm_body,
        out_shape=jax.ShapeDtypeStruct((_B, _CLS), jnp.float32),
    )(pooled, W, b.reshape(1, _CLS))
    return logits
